# initial kernel scaffold (unmeasured)
import jax
import jax.numpy as jnp
from jax import lax
from jax.experimental import pallas as pl
from jax.experimental.pallas import tpu as pltpu

N_DEV = 8


def kernel(x, w_mat, scale_x, scale_w):
    m_per, k = x.shape
    n_per = w_mat.shape[1]

    def body(x_ref, w_ref, sx_ref, sw_ref, out_ref,
             comm_ref, send_sems, recv_sems):
        my = lax.axis_index("i")
        left = lax.rem(my + N_DEV - 1, N_DEV)
        right = lax.rem(my + 1, N_DEV)

        barrier_sem = pltpu.get_barrier_semaphore()
        pl.semaphore_signal(barrier_sem, inc=1, device_id=(left,),
                            device_id_type=pl.DeviceIdType.MESH)
        pl.semaphore_signal(barrier_sem, inc=1, device_id=(right,),
                            device_id_type=pl.DeviceIdType.MESH)
        pl.semaphore_wait(barrier_sem, 2)

        comm_ref[0] = x_ref[...]
        scale = sx_ref[0] * sw_ref[0]

        def gemm_store(slot, origin):
            acc = lax.dot_general(
                comm_ref[slot], w_ref[...],
                dimension_numbers=(((1,), (0,)), ((), ())),
                preferred_element_type=jnp.float32,
            )
            y = jnp.maximum(acc * scale, 0.0)
            out_ref[pl.ds(origin * m_per, m_per), :] = y

        gemm_store(0, my)

        for h in range(N_DEV - 1):
            rdma = pltpu.make_async_remote_copy(
                src_ref=comm_ref.at[h],
                dst_ref=comm_ref.at[h + 1],
                send_sem=send_sems.at[h],
                recv_sem=recv_sems.at[h],
                device_id=(right,),
                device_id_type=pl.DeviceIdType.MESH,
            )
            rdma.start()
            rdma.wait()
            origin = lax.rem(my + N_DEV - 1 - h, N_DEV)
            gemm_store(h + 1, origin)

    return pl.pallas_call(
        body,
        out_shape=jax.ShapeDtypeStruct((N_DEV * m_per, n_per), jnp.float32),
        in_specs=[
            pl.BlockSpec(memory_space=pltpu.VMEM),
            pl.BlockSpec(memory_space=pltpu.VMEM),
            pl.BlockSpec(memory_space=pltpu.SMEM),
            pl.BlockSpec(memory_space=pltpu.SMEM),
        ],
        out_specs=pl.BlockSpec(memory_space=pltpu.VMEM),
        scratch_shapes=[
            pltpu.VMEM((N_DEV, m_per, k), x.dtype),
            pltpu.SemaphoreType.DMA((N_DEV - 1,)),
            pltpu.SemaphoreType.DMA((N_DEV - 1,)),
        ],
        compiler_params=pltpu.CompilerParams(collective_id=0),
    )(x, w_mat, scale_x, scale_w)


# baseline (device time: 221526 ns/iter reference)
import jax
import jax.numpy as jnp
from jax import lax
from jax.experimental import pallas as pl
from jax.experimental.pallas import tpu as pltpu

N_DEV = 8


def kernel(x, w_mat, scale_x, scale_w):
    m_per, k = x.shape
    n_per = w_mat.shape[1]

    def body(x_ref, w_ref, sx_ref, sw_ref, out_ref,
             comm_ref, w8_ref, send_sems, recv_sems):
        my = lax.axis_index("i")
        left = lax.rem(my + N_DEV - 1, N_DEV)
        right = lax.rem(my + 1, N_DEV)

        barrier_sem = pltpu.get_barrier_semaphore()
        pl.semaphore_signal(barrier_sem, inc=1, device_id=(left,),
                            device_id_type=pl.DeviceIdType.MESH)
        pl.semaphore_signal(barrier_sem, inc=1, device_id=(right,),
                            device_id_type=pl.DeviceIdType.MESH)
        pl.semaphore_wait(barrier_sem, 2)

        comm_ref[0] = x_ref[...].astype(jnp.float8_e4m3fn)
        w8_ref[...] = w_ref[...].astype(jnp.float8_e4m3fn)
        scale = sx_ref[0] * sw_ref[0]

        def gemm_store(slot, origin):
            acc = lax.dot_general(
                comm_ref[slot], w8_ref[...],
                dimension_numbers=(((1,), (0,)), ((), ())),
                preferred_element_type=jnp.float32,
            )
            y = jnp.maximum(acc * scale, 0.0)
            out_ref[pl.ds(origin * m_per, m_per), :] = y

        gemm_store(0, my)

        for h in range(N_DEV - 1):
            rdma = pltpu.make_async_remote_copy(
                src_ref=comm_ref.at[h],
                dst_ref=comm_ref.at[h + 1],
                send_sem=send_sems.at[h],
                recv_sem=recv_sems.at[h],
                device_id=(right,),
                device_id_type=pl.DeviceIdType.MESH,
            )
            rdma.start()
            rdma.wait()
            origin = lax.rem(my + N_DEV - 1 - h, N_DEV)
            gemm_store(h + 1, origin)

    return pl.pallas_call(
        body,
        out_shape=jax.ShapeDtypeStruct((N_DEV * m_per, n_per), jnp.float32),
        in_specs=[
            pl.BlockSpec(memory_space=pltpu.VMEM),
            pl.BlockSpec(memory_space=pltpu.VMEM),
            pl.BlockSpec(memory_space=pltpu.SMEM),
            pl.BlockSpec(memory_space=pltpu.SMEM),
        ],
        out_specs=pl.BlockSpec(memory_space=pltpu.VMEM),
        scratch_shapes=[
            pltpu.VMEM((N_DEV, m_per, k), jnp.float8_e4m3fn),
            pltpu.VMEM((k, n_per), jnp.float8_e4m3fn),
            pltpu.SemaphoreType.DMA((N_DEV - 1,)),
            pltpu.SemaphoreType.DMA((N_DEV - 1,)),
        ],
        compiler_params=pltpu.CompilerParams(
            collective_id=0,
            vmem_limit_bytes=63 * 1024 * 1024,
        ),
    )(x, w_mat, scale_x, scale_w)


# device time: 127665 ns/iter; 1.7352x vs baseline; 1.7352x over previous
import jax
import jax.numpy as jnp
from jax import lax
from jax.experimental import pallas as pl
from jax.experimental.pallas import tpu as pltpu

N_DEV = 8


def kernel(x, w_mat, scale_x, scale_w):
    m_per, k = x.shape
    n_per = w_mat.shape[1]
    h_per = m_per // 2

    def body(x_ref, w_ref, sx_ref, sw_ref, out_ref,
             cw_ref, ccw_ref, w8_ref,
             cw_send, cw_recv, ccw_send, ccw_recv):
        my = lax.axis_index("i")
        left = lax.rem(my + N_DEV - 1, N_DEV)
        right = lax.rem(my + 1, N_DEV)

        barrier_sem = pltpu.get_barrier_semaphore()
        pl.semaphore_signal(barrier_sem, inc=1, device_id=(left,),
                            device_id_type=pl.DeviceIdType.MESH)
        pl.semaphore_signal(barrier_sem, inc=1, device_id=(right,),
                            device_id_type=pl.DeviceIdType.MESH)
        pl.semaphore_wait(barrier_sem, 2)

        cw_ref[0] = x_ref[:h_per, :].astype(jnp.float8_e4m3fn)
        ccw_ref[0] = x_ref[h_per:, :].astype(jnp.float8_e4m3fn)

        def hop_rdmas(h):
            cw = pltpu.make_async_remote_copy(
                src_ref=cw_ref.at[h],
                dst_ref=cw_ref.at[h + 1],
                send_sem=cw_send.at[h],
                recv_sem=cw_recv.at[h],
                device_id=(right,),
                device_id_type=pl.DeviceIdType.MESH,
            )
            ccw = pltpu.make_async_remote_copy(
                src_ref=ccw_ref.at[h],
                dst_ref=ccw_ref.at[h + 1],
                send_sem=ccw_send.at[h],
                recv_sem=ccw_recv.at[h],
                device_id=(left,),
                device_id_type=pl.DeviceIdType.MESH,
            )
            return cw, ccw

        rdmas = [hop_rdmas(0)]
        rdmas[0][0].start()
        rdmas[0][1].start()

        w8_ref[...] = w_ref[...].astype(jnp.float8_e4m3fn)
        scale = sx_ref[0] * sw_ref[0]

        def gemm_store(src_ref, slot, row_start):
            acc = lax.dot_general(
                src_ref[slot], w8_ref[...],
                dimension_numbers=(((1,), (0,)), ((), ())),
                preferred_element_type=jnp.float32,
            )
            y = jnp.maximum(acc * scale, 0.0)
            out_ref[pl.ds(row_start, h_per), :] = y

        gemm_store(cw_ref, 0, my * m_per)
        gemm_store(ccw_ref, 0, my * m_per + h_per)

        for h in range(N_DEV - 1):
            cw, ccw = rdmas[h]
            cw.wait_recv()
            ccw.wait_recv()
            if h < N_DEV - 2:
                nxt = hop_rdmas(h + 1)
                nxt[0].start()
                nxt[1].start()
                rdmas.append(nxt)
            o_cw = lax.rem(my + N_DEV - 1 - h, N_DEV)
            o_ccw = lax.rem(my + 1 + h, N_DEV)
            gemm_store(cw_ref, h + 1, o_cw * m_per)
            gemm_store(ccw_ref, h + 1, o_ccw * m_per + h_per)

        for cw, ccw in rdmas:
            cw.wait_send()
            ccw.wait_send()

    return pl.pallas_call(
        body,
        out_shape=jax.ShapeDtypeStruct((N_DEV * m_per, n_per), jnp.float32),
        in_specs=[
            pl.BlockSpec(memory_space=pltpu.VMEM),
            pl.BlockSpec(memory_space=pltpu.VMEM),
            pl.BlockSpec(memory_space=pltpu.SMEM),
            pl.BlockSpec(memory_space=pltpu.SMEM),
        ],
        out_specs=pl.BlockSpec(memory_space=pltpu.VMEM),
        scratch_shapes=[
            pltpu.VMEM((N_DEV, h_per, k), jnp.float8_e4m3fn),
            pltpu.VMEM((N_DEV, h_per, k), jnp.float8_e4m3fn),
            pltpu.VMEM((k, n_per), jnp.float8_e4m3fn),
            pltpu.SemaphoreType.DMA((N_DEV - 1,)),
            pltpu.SemaphoreType.DMA((N_DEV - 1,)),
            pltpu.SemaphoreType.DMA((N_DEV - 1,)),
            pltpu.SemaphoreType.DMA((N_DEV - 1,)),
        ],
        compiler_params=pltpu.CompilerParams(
            collective_id=0,
            vmem_limit_bytes=63 * 1024 * 1024,
        ),
    )(x, w_mat, scale_x, scale_w)


# device time: 124473 ns/iter; 1.7797x vs baseline; 1.0256x over previous
import jax
import jax.numpy as jnp
from jax import lax
from jax.experimental import pallas as pl
from jax.experimental.pallas import tpu as pltpu

N_DEV = 8


def kernel(x, w_mat, scale_x, scale_w):
    m_per, k = x.shape
    n_per = w_mat.shape[1]
    h_per = m_per // 2

    def body(x_ref, w_ref, sx_ref, sw_ref, out_ref,
             cw_ref, ccw_ref, w8_ref,
             cw_send, cw_recv, ccw_send, ccw_recv):
        my = lax.axis_index("i")

        def ring_to_pos(rr):
            m = rr // 4
            return rr + m * (11 - 2 * rr)

        r = ring_to_pos(my)
        right = ring_to_pos(lax.rem(r + 1, N_DEV))
        left = ring_to_pos(lax.rem(r + N_DEV - 1, N_DEV))

        barrier_sem = pltpu.get_barrier_semaphore()
        pl.semaphore_signal(barrier_sem, inc=1, device_id=(left,),
                            device_id_type=pl.DeviceIdType.MESH)
        pl.semaphore_signal(barrier_sem, inc=1, device_id=(right,),
                            device_id_type=pl.DeviceIdType.MESH)
        pl.semaphore_wait(barrier_sem, 2)

        cw_ref[0] = x_ref[:h_per, :].astype(jnp.float8_e4m3fn)
        ccw_ref[0] = x_ref[h_per:, :].astype(jnp.float8_e4m3fn)

        def hop_rdmas(h):
            cw = pltpu.make_async_remote_copy(
                src_ref=cw_ref.at[h],
                dst_ref=cw_ref.at[h + 1],
                send_sem=cw_send.at[h],
                recv_sem=cw_recv.at[h],
                device_id=(right,),
                device_id_type=pl.DeviceIdType.MESH,
            )
            ccw = pltpu.make_async_remote_copy(
                src_ref=ccw_ref.at[h],
                dst_ref=ccw_ref.at[h + 1],
                send_sem=ccw_send.at[h],
                recv_sem=ccw_recv.at[h],
                device_id=(left,),
                device_id_type=pl.DeviceIdType.MESH,
            )
            return cw, ccw

        rdmas = [hop_rdmas(0)]
        rdmas[0][0].start()
        rdmas[0][1].start()

        w8_ref[...] = w_ref[...].astype(jnp.float8_e4m3fn)
        scale = sx_ref[0] * sw_ref[0]

        def gemm_store(src_ref, slot, row_start):
            acc = lax.dot_general(
                src_ref[slot], w8_ref[...],
                dimension_numbers=(((1,), (0,)), ((), ())),
                preferred_element_type=jnp.float32,
            )
            y = jnp.maximum(acc * scale, 0.0)
            out_ref[pl.ds(row_start, h_per), :] = y

        gemm_store(cw_ref, 0, my * m_per)
        gemm_store(ccw_ref, 0, my * m_per + h_per)

        for h in range(N_DEV - 1):
            cw, ccw = rdmas[h]
            cw.wait_recv()
            ccw.wait_recv()
            if h < N_DEV - 2:
                nxt = hop_rdmas(h + 1)
                nxt[0].start()
                nxt[1].start()
                rdmas.append(nxt)
            o_cw = ring_to_pos(lax.rem(r + N_DEV - 1 - h, N_DEV))
            o_ccw = ring_to_pos(lax.rem(r + 1 + h, N_DEV))
            gemm_store(cw_ref, h + 1, o_cw * m_per)
            gemm_store(ccw_ref, h + 1, o_ccw * m_per + h_per)

        for cw, ccw in rdmas:
            cw.wait_send()
            ccw.wait_send()

    return pl.pallas_call(
        body,
        out_shape=jax.ShapeDtypeStruct((N_DEV * m_per, n_per), jnp.float32),
        in_specs=[
            pl.BlockSpec(memory_space=pltpu.VMEM),
            pl.BlockSpec(memory_space=pltpu.VMEM),
            pl.BlockSpec(memory_space=pltpu.SMEM),
            pl.BlockSpec(memory_space=pltpu.SMEM),
        ],
        out_specs=pl.BlockSpec(memory_space=pltpu.VMEM),
        scratch_shapes=[
            pltpu.VMEM((N_DEV, h_per, k), jnp.float8_e4m3fn),
            pltpu.VMEM((N_DEV, h_per, k), jnp.float8_e4m3fn),
            pltpu.VMEM((k, n_per), jnp.float8_e4m3fn),
            pltpu.SemaphoreType.DMA((N_DEV - 1,)),
            pltpu.SemaphoreType.DMA((N_DEV - 1,)),
            pltpu.SemaphoreType.DMA((N_DEV - 1,)),
            pltpu.SemaphoreType.DMA((N_DEV - 1,)),
        ],
        compiler_params=pltpu.CompilerParams(
            collective_id=0,
            vmem_limit_bytes=63 * 1024 * 1024,
        ),
    )(x, w_mat, scale_x, scale_w)


# device time: 106082 ns/iter; 2.0883x vs baseline; 1.1734x over previous
import jax
import jax.numpy as jnp
from jax import lax
from jax.experimental import pallas as pl
from jax.experimental.pallas import tpu as pltpu

N_DEV = 8

S1A, S1B, S1C, S2A, S2B, S2C, S3 = range(7)


def kernel(x, w_mat, scale_x, scale_w):
    m_per, k = x.shape
    n_per = w_mat.shape[1]

    def body(x_ref, w_ref, sx_ref, sw_ref, out_ref,
             own8, l1, l2, r1b, r2b, c1, c2, c3, w8_ref,
             send_sems, recv_sems):
        my = lax.axis_index("i")

        def ring_to_pos(rr):
            mm = rr // 4
            return rr + mm * (11 - 2 * rr)

        r = ring_to_pos(my)
        right = ring_to_pos(lax.rem(r + 1, N_DEV))
        left = ring_to_pos(lax.rem(r + N_DEV - 1, N_DEV))
        parity = lax.rem(r, 2)
        sign = 1 - 2 * parity
        partner_r = lax.rem(r + N_DEV + 3 * sign, N_DEV)
        partner = ring_to_pos(partner_r)
        is_even = parity == 0
        is_odd = parity == 1

        barrier_sem = pltpu.get_barrier_semaphore()
        for nbr in (left, right, partner):
            pl.semaphore_signal(barrier_sem, inc=1, device_id=(nbr,),
                                device_id_type=pl.DeviceIdType.MESH)
        pl.semaphore_wait(barrier_sem, 3)

        own8[...] = x_ref[...].astype(jnp.float8_e4m3fn)

        def rdma(src, dst, sem_idx, target):
            return pltpu.make_async_remote_copy(
                src_ref=src, dst_ref=dst,
                send_sem=send_sems.at[sem_idx],
                recv_sem=recv_sems.at[sem_idx],
                device_id=(target,),
                device_id_type=pl.DeviceIdType.MESH,
            )

        s1a = rdma(own8, l1, S1A, right)
        s1b = rdma(own8, r1b, S1B, left)
        s1c = rdma(own8, c1, S1C, partner)
        s1a.start()
        s1b.start()
        s1c.start()

        w8_ref[...] = w_ref[...].astype(jnp.float8_e4m3fn)
        scale = sx_ref[0] * sw_ref[0]

        def gemm_store(src_ref, origin_pos):
            acc = lax.dot_general(
                src_ref[...], w8_ref[...],
                dimension_numbers=(((1,), (0,)), ((), ())),
                preferred_element_type=jnp.float32,
            )
            y = jnp.maximum(acc * scale, 0.0)
            out_ref[pl.ds(origin_pos * m_per, m_per), :] = y

        gemm_store(own8, my)

        s1a.wait_recv()
        s1b.wait_recv()

        s2a = rdma(l1, l2, S2A, right)
        s2b = rdma(r1b, r2b, S2B, left)
        s2a.start()
        s2b.start()
        s2c_even = rdma(l1, c2, S2C, partner)
        s2c_odd = rdma(r1b, c2, S2C, partner)

        @pl.when(is_even)
        def _():
            s2c_even.start()

        @pl.when(is_odd)
        def _():
            s2c_odd.start()

        gemm_store(l1, ring_to_pos(lax.rem(r + N_DEV - 1, N_DEV)))
        gemm_store(r1b, ring_to_pos(lax.rem(r + 1, N_DEV)))

        s1c.wait_recv()
        gemm_store(c1, partner)

        s2a.wait_recv()
        s2b.wait_recv()

        s3_even = rdma(l2, c3, S3, partner)
        s3_odd = rdma(r2b, c3, S3, partner)

        @pl.when(is_even)
        def _():
            s3_even.start()

        @pl.when(is_odd)
        def _():
            s3_odd.start()

        gemm_store(l2, ring_to_pos(lax.rem(r + N_DEV - 2, N_DEV)))
        gemm_store(r2b, ring_to_pos(lax.rem(r + 2, N_DEV)))

        s2c_even.wait_recv()
        gemm_store(c2, ring_to_pos(lax.rem(r + 4, N_DEV)))

        s3_even.wait_recv()
        gemm_store(c3, ring_to_pos(lax.rem(r + N_DEV - 3 * sign, N_DEV)))

        for d in (s1a, s1b, s1c, s2a, s2b, s2c_even, s3_even):
            d.wait_send()

    chunk = pltpu.VMEM((m_per, k), jnp.float8_e4m3fn)
    return pl.pallas_call(
        body,
        out_shape=jax.ShapeDtypeStruct((N_DEV * m_per, n_per), jnp.float32),
        in_specs=[
            pl.BlockSpec(memory_space=pltpu.VMEM),
            pl.BlockSpec(memory_space=pltpu.VMEM),
            pl.BlockSpec(memory_space=pltpu.SMEM),
            pl.BlockSpec(memory_space=pltpu.SMEM),
        ],
        out_specs=pl.BlockSpec(memory_space=pltpu.VMEM),
        scratch_shapes=[
            chunk,
            chunk,
            chunk,
            chunk,
            chunk,
            chunk,
            chunk,
            chunk,
            pltpu.VMEM((k, n_per), jnp.float8_e4m3fn),
            pltpu.SemaphoreType.DMA((7,)),
            pltpu.SemaphoreType.DMA((7,)),
        ],
        compiler_params=pltpu.CompilerParams(
            collective_id=0,
            vmem_limit_bytes=63 * 1024 * 1024,
        ),
    )(x, w_mat, scale_x, scale_w)


# device time: 94318 ns/iter; 2.3487x vs baseline; 1.1247x over previous
import jax
import jax.numpy as jnp
from jax import lax
from jax.experimental import pallas as pl
from jax.experimental.pallas import tpu as pltpu

N_DEV = 8

S1A, S1B, S1C, S2A, S2B, S2C, S3C, S3R = range(8)


def kernel(x, w_mat, scale_x, scale_w):
    m_per, k = x.shape
    n_per = w_mat.shape[1]
    half = m_per // 2

    def body(x_ref, w_ref, sx_ref, sw_ref, out_ref,
             own8, l1, l2, r1b, r2b, c1, c2, c3h, r3h, w8_ref,
             send_sems, recv_sems):
        my = lax.axis_index("i")

        def ring_to_pos(rr):
            mm = rr // 4
            return rr + mm * (11 - 2 * rr)

        r = ring_to_pos(my)
        right = ring_to_pos(lax.rem(r + 1, N_DEV))
        left = ring_to_pos(lax.rem(r + N_DEV - 1, N_DEV))
        parity = lax.rem(r, 2)
        sign = 1 - 2 * parity
        partner_r = lax.rem(r + N_DEV + 3 * sign, N_DEV)
        partner = ring_to_pos(partner_r)
        is_even = parity == 0
        is_odd = parity == 1

        barrier_sem = pltpu.get_barrier_semaphore()
        for nbr in (left, right, partner):
            pl.semaphore_signal(barrier_sem, inc=1, device_id=(nbr,),
                                device_id_type=pl.DeviceIdType.MESH)
        pl.semaphore_wait(barrier_sem, 3)

        own8[...] = x_ref[...].astype(jnp.float8_e4m3fn)

        def rdma(src, dst, sem_idx, target):
            return pltpu.make_async_remote_copy(
                src_ref=src, dst_ref=dst,
                send_sem=send_sems.at[sem_idx],
                recv_sem=recv_sems.at[sem_idx],
                device_id=(target,),
                device_id_type=pl.DeviceIdType.MESH,
            )

        s1a = rdma(own8, l1, S1A, right)
        s1b = rdma(own8, r1b, S1B, left)
        s1c = rdma(own8, c1, S1C, partner)
        s1a.start()
        s1b.start()
        s1c.start()

        w8_ref[...] = w_ref[...].astype(jnp.float8_e4m3fn)
        scale = sx_ref[0] * sw_ref[0]

        def gemm_store(src_ref, origin_pos, rows, row_off):
            acc = lax.dot_general(
                src_ref[...], w8_ref[...],
                dimension_numbers=(((1,), (0,)), ((), ())),
                preferred_element_type=jnp.float32,
            )
            y = jnp.maximum(acc * scale, 0.0)
            out_ref[pl.ds(origin_pos * m_per + row_off, rows), :] = y

        gemm_store(own8, my, m_per, 0)

        s1a.wait_recv()
        s2a = rdma(l1, l2, S2A, right)
        s2a.start()
        s2c_even = rdma(l1, c2, S2C, partner)

        @pl.when(is_even)
        def _():
            s2c_even.start()

        s1b.wait_recv()
        s2b = rdma(r1b, r2b, S2B, left)
        s2b.start()
        s2c_odd = rdma(r1b, c2, S2C, partner)

        @pl.when(is_odd)
        def _():
            s2c_odd.start()

        gemm_store(l1, ring_to_pos(lax.rem(r + N_DEV - 1, N_DEV)), m_per, 0)
        gemm_store(r1b, ring_to_pos(lax.rem(r + 1, N_DEV)), m_per, 0)

        s1c.wait_recv()
        gemm_store(c1, partner, m_per, 0)

        s2a.wait_recv()
        s3c_even = rdma(l2.at[:half], c3h, S3C, partner)
        s3r_odd = rdma(l2.at[half:], r3h, S3R, right)

        @pl.when(is_even)
        def _():
            s3c_even.start()

        @pl.when(is_odd)
        def _():
            s3r_odd.start()

        s2b.wait_recv()
        s3c_odd = rdma(r2b.at[:half], c3h, S3C, partner)
        s3r_even = rdma(r2b.at[half:], r3h, S3R, left)

        @pl.when(is_odd)
        def _():
            s3c_odd.start()

        @pl.when(is_even)
        def _():
            s3r_even.start()

        gemm_store(l2, ring_to_pos(lax.rem(r + N_DEV - 2, N_DEV)), m_per, 0)
        gemm_store(r2b, ring_to_pos(lax.rem(r + 2, N_DEV)), m_per, 0)

        s2c_even.wait_recv()
        gemm_store(c2, ring_to_pos(lax.rem(r + 4, N_DEV)), m_per, 0)

        o3 = ring_to_pos(lax.rem(r + N_DEV - 3 * sign, N_DEV))
        s3c_even.wait_recv()
        gemm_store(c3h, o3, half, 0)
        s3r_odd.wait_recv()
        gemm_store(r3h, o3, half, half)

        for d in (s1a, s1b, s1c, s2a, s2b, s2c_even, s3c_even, s3r_even):
            d.wait_send()

    chunk = pltpu.VMEM((m_per, k), jnp.float8_e4m3fn)
    halfchunk = pltpu.VMEM((half, k), jnp.float8_e4m3fn)
    return pl.pallas_call(
        body,
        out_shape=jax.ShapeDtypeStruct((N_DEV * m_per, n_per), jnp.float32),
        in_specs=[
            pl.BlockSpec(memory_space=pltpu.VMEM),
            pl.BlockSpec(memory_space=pltpu.VMEM),
            pl.BlockSpec(memory_space=pltpu.SMEM),
            pl.BlockSpec(memory_space=pltpu.SMEM),
        ],
        out_specs=pl.BlockSpec(memory_space=pltpu.VMEM),
        scratch_shapes=[
            chunk,
            chunk,
            chunk,
            chunk,
            chunk,
            chunk,
            chunk,
            halfchunk,
            halfchunk,
            pltpu.VMEM((k, n_per), jnp.float8_e4m3fn),
            pltpu.SemaphoreType.DMA((8,)),
            pltpu.SemaphoreType.DMA((8,)),
        ],
        compiler_params=pltpu.CompilerParams(
            collective_id=0,
            vmem_limit_bytes=63 * 1024 * 1024,
        ),
    )(x, w_mat, scale_x, scale_w)


# device time: 92411 ns/iter; 2.3972x vs baseline; 1.0206x over previous
import jax
import jax.numpy as jnp
from jax import lax
from jax.experimental import pallas as pl
from jax.experimental.pallas import tpu as pltpu

N_DEV = 8

(S1A1, S1A2, S1B1, S1B2, S1C,
 S2A1, S2A2, S2B1, S2B2, S2C1, S2C2,
 S3C, S3R) = range(13)


def kernel(x, w_mat, scale_x, scale_w):
    m_per, k = x.shape
    n_per = w_mat.shape[1]
    half = m_per // 2

    def body(x_ref, w_ref, sx_ref, sw_ref, out_ref,
             own8, l1, l2, r1b, r2b, c1, c2, c3h, r3h, w8_ref,
             send_sems, recv_sems):
        my = lax.axis_index("i")

        def ring_to_pos(rr):
            mm = rr // 4
            return rr + mm * (11 - 2 * rr)

        r = ring_to_pos(my)
        right = ring_to_pos(lax.rem(r + 1, N_DEV))
        left = ring_to_pos(lax.rem(r + N_DEV - 1, N_DEV))
        parity = lax.rem(r, 2)
        sign = 1 - 2 * parity
        partner_r = lax.rem(r + N_DEV + 3 * sign, N_DEV)
        partner = ring_to_pos(partner_r)
        is_even = parity == 0
        is_odd = parity == 1

        barrier_sem = pltpu.get_barrier_semaphore()
        for nbr in (left, right, partner):
            pl.semaphore_signal(barrier_sem, inc=1, device_id=(nbr,),
                                device_id_type=pl.DeviceIdType.MESH)
        pl.semaphore_wait(barrier_sem, 3)

        own8[...] = x_ref[...].astype(jnp.float8_e4m3fn)

        def rdma(src, dst, sem_idx, target):
            return pltpu.make_async_remote_copy(
                src_ref=src, dst_ref=dst,
                send_sem=send_sems.at[sem_idx],
                recv_sem=recv_sems.at[sem_idx],
                device_id=(target,),
                device_id_type=pl.DeviceIdType.MESH,
            )

        def top(ref):
            return ref.at[:half]

        def bot(ref):
            return ref.at[half:]

        s1a1 = rdma(top(own8), top(l1), S1A1, right)
        s1a2 = rdma(bot(own8), bot(l1), S1A2, right)
        s1b1 = rdma(top(own8), top(r1b), S1B1, left)
        s1b2 = rdma(bot(own8), bot(r1b), S1B2, left)
        s1c = rdma(own8, c1, S1C, partner)
        s1a1.start()
        s1b1.start()
        s1c.start()
        s1a2.start()
        s1b2.start()

        w8_ref[...] = w_ref[...].astype(jnp.float8_e4m3fn)
        scale = sx_ref[0] * sw_ref[0]

        def gemm_store(src_ref, origin_pos, rows, row_off):
            acc = lax.dot_general(
                src_ref[...], w8_ref[...],
                dimension_numbers=(((1,), (0,)), ((), ())),
                preferred_element_type=jnp.float32,
            )
            y = jnp.maximum(acc * scale, 0.0)
            out_ref[pl.ds(origin_pos * m_per + row_off, rows), :] = y

        gemm_store(own8, my, m_per, 0)

        s2a1 = rdma(top(l1), top(l2), S2A1, right)
        s2a2 = rdma(bot(l1), bot(l2), S2A2, right)
        s2b1 = rdma(top(r1b), top(r2b), S2B1, left)
        s2b2 = rdma(bot(r1b), bot(r2b), S2B2, left)
        s2c1_e = rdma(top(l1), top(c2), S2C1, partner)
        s2c2_e = rdma(bot(l1), bot(c2), S2C2, partner)
        s2c1_o = rdma(top(r1b), top(c2), S2C1, partner)
        s2c2_o = rdma(bot(r1b), bot(c2), S2C2, partner)

        s1a1.wait_recv()
        s2a1.start()

        @pl.when(is_even)
        def _():
            s2c1_e.start()

        s1b1.wait_recv()
        s2b1.start()

        @pl.when(is_odd)
        def _():
            s2c1_o.start()

        s1a2.wait_recv()
        s2a2.start()

        @pl.when(is_even)
        def _():
            s2c2_e.start()

        s1b2.wait_recv()
        s2b2.start()

        @pl.when(is_odd)
        def _():
            s2c2_o.start()

        gemm_store(l1, ring_to_pos(lax.rem(r + N_DEV - 1, N_DEV)), m_per, 0)
        gemm_store(r1b, ring_to_pos(lax.rem(r + 1, N_DEV)), m_per, 0)

        s1c.wait_recv()
        gemm_store(c1, partner, m_per, 0)

        s3c_e = rdma(top(l2), c3h, S3C, partner)
        s3c_o = rdma(top(r2b), c3h, S3C, partner)
        s3r_o = rdma(bot(l2), r3h, S3R, right)
        s3r_e = rdma(bot(r2b), r3h, S3R, left)

        s2a1.wait_recv()

        @pl.when(is_even)
        def _():
            s3c_e.start()

        s2b1.wait_recv()

        @pl.when(is_odd)
        def _():
            s3c_o.start()

        s2a2.wait_recv()

        @pl.when(is_odd)
        def _():
            s3r_o.start()

        s2b2.wait_recv()

        @pl.when(is_even)
        def _():
            s3r_e.start()

        gemm_store(l2, ring_to_pos(lax.rem(r + N_DEV - 2, N_DEV)), m_per, 0)
        gemm_store(r2b, ring_to_pos(lax.rem(r + 2, N_DEV)), m_per, 0)

        s2c1_e.wait_recv()
        s2c2_e.wait_recv()
        gemm_store(c2, ring_to_pos(lax.rem(r + 4, N_DEV)), m_per, 0)

        o3 = ring_to_pos(lax.rem(r + N_DEV - 3 * sign, N_DEV))
        s3c_e.wait_recv()
        gemm_store(c3h, o3, half, 0)
        s3r_e.wait_recv()
        gemm_store(r3h, o3, half, half)

        for d in (s1a1, s1a2, s1b1, s1b2, s1c,
                  s2a1, s2a2, s2b1, s2b2, s2c1_e, s2c2_e,
                  s3c_e, s3r_e):
            d.wait_send()

    chunk = pltpu.VMEM((m_per, k), jnp.float8_e4m3fn)
    halfchunk = pltpu.VMEM((half, k), jnp.float8_e4m3fn)
    return pl.pallas_call(
        body,
        out_shape=jax.ShapeDtypeStruct((N_DEV * m_per, n_per), jnp.float32),
        in_specs=[
            pl.BlockSpec(memory_space=pltpu.VMEM),
            pl.BlockSpec(memory_space=pltpu.VMEM),
            pl.BlockSpec(memory_space=pltpu.SMEM),
            pl.BlockSpec(memory_space=pltpu.SMEM),
        ],
        out_specs=pl.BlockSpec(memory_space=pltpu.VMEM),
        scratch_shapes=[
            chunk,
            chunk,
            chunk,
            chunk,
            chunk,
            chunk,
            chunk,
            halfchunk,
            halfchunk,
            pltpu.VMEM((k, n_per), jnp.float8_e4m3fn),
            pltpu.SemaphoreType.DMA((13,)),
            pltpu.SemaphoreType.DMA((13,)),
        ],
        compiler_params=pltpu.CompilerParams(
            collective_id=0,
            vmem_limit_bytes=63 * 1024 * 1024,
        ),
    )(x, w_mat, scale_x, scale_w)


# device time: 92400 ns/iter; 2.3975x vs baseline; 1.0001x over previous
import jax
import jax.numpy as jnp
from jax import lax
from jax.experimental import pallas as pl
from jax.experimental.pallas import tpu as pltpu

N_DEV = 8
NQ = 4

S1A, S1B, S1C, S2A, S2B, S2C = 0, 4, 8, 12, 16, 20
S3C, S3R = 24, 25


def kernel(x, w_mat, scale_x, scale_w):
    m_per, k = x.shape
    n_per = w_mat.shape[1]
    half = m_per // 2
    qrow = m_per // NQ

    def body(x_ref, w_ref, sx_ref, sw_ref, out_ref,
             own8, l1, l2, r1b, r2b, c1, c2, c3h, r3h, w8_ref,
             send_sems, recv_sems):
        my = lax.axis_index("i")

        def ring_to_pos(rr):
            mm = rr // 4
            return rr + mm * (11 - 2 * rr)

        r = ring_to_pos(my)
        right = ring_to_pos(lax.rem(r + 1, N_DEV))
        left = ring_to_pos(lax.rem(r + N_DEV - 1, N_DEV))
        parity = lax.rem(r, 2)
        sign = 1 - 2 * parity
        partner_r = lax.rem(r + N_DEV + 3 * sign, N_DEV)
        partner = ring_to_pos(partner_r)
        is_even = parity == 0
        is_odd = parity == 1

        barrier_sem = pltpu.get_barrier_semaphore()
        for nbr in (left, right, partner):
            pl.semaphore_signal(barrier_sem, inc=1, device_id=(nbr,),
                                device_id_type=pl.DeviceIdType.MESH)
        pl.semaphore_wait(barrier_sem, 3)

        def quarter(ref, q):
            return ref.at[q * qrow:(q + 1) * qrow]

        def rdma(src, dst, sem_idx, target):
            return pltpu.make_async_remote_copy(
                src_ref=src, dst_ref=dst,
                send_sem=send_sems.at[sem_idx],
                recv_sem=recv_sems.at[sem_idx],
                device_id=(target,),
                device_id_type=pl.DeviceIdType.MESH,
            )

        s1a, s1b, s1c = [], [], []
        for q in range(NQ):
            own8[q * qrow:(q + 1) * qrow] = (
                x_ref[q * qrow:(q + 1) * qrow].astype(jnp.float8_e4m3fn))
            a = rdma(quarter(own8, q), quarter(l1, q), S1A + q, right)
            b = rdma(quarter(own8, q), quarter(r1b, q), S1B + q, left)
            c = rdma(quarter(own8, q), quarter(c1, q), S1C + q, partner)
            a.start()
            b.start()
            c.start()
            s1a.append(a)
            s1b.append(b)
            s1c.append(c)

        w8_ref[...] = w_ref[...].astype(jnp.float8_e4m3fn)
        scale = sx_ref[0] * sw_ref[0]

        def gemm_store(src_ref, origin_pos, rows, row_off):
            acc = lax.dot_general(
                src_ref[...], w8_ref[...],
                dimension_numbers=(((1,), (0,)), ((), ())),
                preferred_element_type=jnp.float32,
            )
            y = jnp.maximum(acc * scale, 0.0)
            out_ref[pl.ds(origin_pos * m_per + row_off, rows), :] = y

        gemm_store(own8, my, m_per, 0)

        s2a, s2b, s2c = [], [], []
        for q in range(NQ):
            a = rdma(quarter(l1, q), quarter(l2, q), S2A + q, right)
            b = rdma(quarter(r1b, q), quarter(r2b, q), S2B + q, left)
            ce = rdma(quarter(l1, q), quarter(c2, q), S2C + q, partner)
            co = rdma(quarter(r1b, q), quarter(c2, q), S2C + q, partner)

            s1a[q].wait_recv()
            a.start()

            @pl.when(is_even)
            def _():
                ce.start()

            s1b[q].wait_recv()
            b.start()

            @pl.when(is_odd)
            def _():
                co.start()

            s2a.append(a)
            s2b.append(b)
            s2c.append(ce)

        gemm_store(l1, ring_to_pos(lax.rem(r + N_DEV - 1, N_DEV)), m_per, 0)
        gemm_store(r1b, ring_to_pos(lax.rem(r + 1, N_DEV)), m_per, 0)

        for q in range(NQ):
            s1c[q].wait_recv()
        gemm_store(c1, partner, m_per, 0)

        s3c_e = rdma(l2.at[:half], c3h, S3C, partner)
        s3c_o = rdma(r2b.at[:half], c3h, S3C, partner)
        s3r_o = rdma(l2.at[half:], r3h, S3R, right)
        s3r_e = rdma(r2b.at[half:], r3h, S3R, left)

        s2a[0].wait_recv()
        s2a[1].wait_recv()

        @pl.when(is_even)
        def _():
            s3c_e.start()

        s2b[0].wait_recv()
        s2b[1].wait_recv()

        @pl.when(is_odd)
        def _():
            s3c_o.start()

        s2a[2].wait_recv()
        s2a[3].wait_recv()

        @pl.when(is_odd)
        def _():
            s3r_o.start()

        s2b[2].wait_recv()
        s2b[3].wait_recv()

        @pl.when(is_even)
        def _():
            s3r_e.start()

        gemm_store(l2, ring_to_pos(lax.rem(r + N_DEV - 2, N_DEV)), m_per, 0)
        gemm_store(r2b, ring_to_pos(lax.rem(r + 2, N_DEV)), m_per, 0)

        for q in range(NQ):
            s2c[q].wait_recv()
        gemm_store(c2, ring_to_pos(lax.rem(r + 4, N_DEV)), m_per, 0)

        o3 = ring_to_pos(lax.rem(r + N_DEV - 3 * sign, N_DEV))
        s3c_e.wait_recv()
        gemm_store(c3h, o3, half, 0)
        s3r_e.wait_recv()
        gemm_store(r3h, o3, half, half)

        for d in s1a + s1b + s1c + s2a + s2b + s2c + [s3c_e, s3r_e]:
            d.wait_send()

    chunk = pltpu.VMEM((m_per, k), jnp.float8_e4m3fn)
    halfchunk = pltpu.VMEM((half, k), jnp.float8_e4m3fn)
    return pl.pallas_call(
        body,
        out_shape=jax.ShapeDtypeStruct((N_DEV * m_per, n_per), jnp.float32),
        in_specs=[
            pl.BlockSpec(memory_space=pltpu.VMEM),
            pl.BlockSpec(memory_space=pltpu.VMEM),
            pl.BlockSpec(memory_space=pltpu.SMEM),
            pl.BlockSpec(memory_space=pltpu.SMEM),
        ],
        out_specs=pl.BlockSpec(memory_space=pltpu.VMEM),
        scratch_shapes=[
            chunk,
            chunk,
            chunk,
            chunk,
            chunk,
            chunk,
            chunk,
            halfchunk,
            halfchunk,
            pltpu.VMEM((k, n_per), jnp.float8_e4m3fn),
            pltpu.SemaphoreType.DMA((26,)),
            pltpu.SemaphoreType.DMA((26,)),
        ],
        compiler_params=pltpu.CompilerParams(
            collective_id=0,
            vmem_limit_bytes=63 * 1024 * 1024,
        ),
    )(x, w_mat, scale_x, scale_w)
